# Initial kernel scaffold; baseline (speedup 1.0000x reference)
import jax
import jax.numpy as jnp
from jax.experimental import pallas as pl


def kernel(hidden, labels, wte_table, wpe_table):
    raise NotImplementedError("write your pallas kernel here")


# speedup vs baseline: 1.2475x; 1.2475x over previous
import functools

import jax
import jax.numpy as jnp
from jax import lax
from jax.experimental import pallas as pl
from jax.experimental.pallas import tpu as pltpu
from jax.experimental.pallas import tpu_sc as plsc

_START_ID = 2
_CNK = 64
_LANES = 16


def _build_emb_kernel(N, D, T, n_workers):
    per_w = N // n_workers
    n_chunks = per_w // _CNK
    mesh = plsc.VectorSubcoreMesh(core_axis_name="c", subcore_axis_name="s")

    @functools.partial(
        pl.kernel,
        mesh=mesh,
        out_type=jax.ShapeDtypeStruct((N, D), jnp.float32),
        scratch_types=[
            pltpu.VMEM((_CNK + 16,), jnp.int32),
            pltpu.VMEM((_CNK,), jnp.int32),
            pltpu.VMEM((_CNK, D), jnp.float32),
            pltpu.VMEM((_CNK, D), jnp.float32),
            pltpu.SemaphoreType.DMA,
        ],
    )
    def emb(lab_hbm, wte_hbm, wpe_hbm, out_hbm, buf, idx, rows, wrows, sem):
        wid = lax.axis_index("s") * 2 + lax.axis_index("c")
        lane = lax.iota(jnp.int32, _LANES)
        for c in range(n_chunks):
            base = wid * per_w + c * _CNK
            pos0 = base % T
            start = (pos0 == 0).astype(jnp.int32)
            pltpu.sync_copy(
                lab_hbm.at[pl.ds(base - 8 + 8 * start, _CNK + 8)],
                buf.at[pl.ds(8 * start, _CNK + 8)],
            )
            for k in range(_CNK // _LANES):
                v = buf[pl.ds(7 + k * _LANES, _LANES)]
                if k == 0:
                    keep = 1 - (1 - jnp.minimum(lane, 1)) * start
                    v = v * keep + _START_ID * (1 - keep)
                idx[pl.ds(k * _LANES, _LANES)] = v
            gat = pltpu.async_copy(wte_hbm.at[idx], rows, sem)
            pltpu.sync_copy(wpe_hbm.at[pl.ds(pos0, _CNK)], wrows)
            gat.wait()

            def add_row(r, carry):
                for k in range(D // _LANES):
                    sl = pl.ds(k * _LANES, _LANES)
                    rows[r, sl] = rows[r, sl] + wrows[r, sl]
                return carry

            lax.fori_loop(0, _CNK, add_row, 0)
            pltpu.sync_copy(rows, out_hbm.at[pl.ds(base, _CNK)])

    return emb


def kernel(hidden, labels, wte_table, wpe_table):
    B, T = labels.shape
    D = wte_table.shape[1]
    N = B * T
    info = plsc.get_sparse_core_info()
    n_workers = info.num_cores * info.num_subcores
    emb = _build_emb_kernel(N, D, T, n_workers)
    out_flat = emb(labels.reshape(N), wte_table, wpe_table)
    return (hidden, out_flat.reshape(B, T, D), labels)
